# Initial kernel scaffold; baseline (speedup 1.0000x reference)
import jax
import jax.numpy as jnp
from jax.experimental import pallas as pl


def kernel(x, edge_index, W1, b1, G1, gb1, W2, b2, G2, gb2):
    raise NotImplementedError("write your pallas kernel here")


# speedup vs baseline: 10.4661x; 10.4661x over previous
import functools

import jax
import jax.numpy as jnp
from jax import lax
from jax.experimental import pallas as pl
from jax.experimental.pallas import tpu as pltpu
from jax.experimental.pallas import tpu_sc as plsc

NC = 2
NS = 16
NW = NC * NS
K = 128
PAD_ROWS = 8

_MESH = dict(core_axis_name="c", subcore_axis_name="s")


@functools.lru_cache(maxsize=None)
def _make_deg_kernel(n, d, nchunk):
    rp = n // NS

    @functools.partial(
        pl.kernel,
        out_type=jax.ShapeDtypeStruct((NC, NS, rp, d), jnp.float32),
        mesh=plsc.VectorSubcoreMesh(**_MESH),
        scratch_types=[
            pltpu.VMEM((nchunk, K), jnp.int32),
            pltpu.VMEM((K, d), jnp.float32),
            pltpu.VMEM_SHARED((n + PAD_ROWS, d), jnp.float32),
        ],
    )
    def deg_kernel(row_hbm, ones_hbm, zeros_hbm, out_hbm, idx_v, ones_v, acc_sh):
        cid = lax.axis_index("c")
        sid = lax.axis_index("s")
        wid = cid * NS + sid
        pltpu.sync_copy(ones_hbm, ones_v)
        pltpu.sync_copy(zeros_hbm, acc_sh.at[pl.ds(sid * rp, rp)])
        pltpu.sync_copy(row_hbm.at[wid], idx_v)
        plsc.subcore_barrier()

        @pl.loop(0, nchunk)
        def _(j):
            pltpu.sync_copy(ones_v, acc_sh.at[idx_v.at[j]], add=True)

        plsc.subcore_barrier()
        pltpu.sync_copy(acc_sh.at[pl.ds(sid * rp, rp)], out_hbm.at[cid, sid])

    return deg_kernel


@functools.lru_cache(maxsize=None)
def _make_edge_kernel(n, d, nchunk):
    rp = n // NS

    @functools.partial(
        pl.kernel,
        out_type=jax.ShapeDtypeStruct((NC, NS, rp, d), jnp.float32),
        mesh=plsc.VectorSubcoreMesh(**_MESH),
        scratch_types=[
            pltpu.VMEM((nchunk, K), jnp.int32),
            pltpu.VMEM((nchunk, K), jnp.int32),
            pltpu.VMEM((K, d), jnp.float32),
            pltpu.VMEM_SHARED((n + PAD_ROWS, d), jnp.float32),
            pltpu.SemaphoreType.DMA,
        ],
    )
    def edge_kernel(hq_hbm, row_hbm, col_hbm, zeros_hbm, out_hbm,
                    ridx_v, cidx_v, rows_v, acc_sh, sem):
        cid = lax.axis_index("c")
        sid = lax.axis_index("s")
        wid = cid * NS + sid
        pltpu.sync_copy(zeros_hbm, acc_sh.at[pl.ds(sid * rp, rp)])
        pltpu.sync_copy(row_hbm.at[wid], ridx_v)
        pltpu.sync_copy(col_hbm.at[wid], cidx_v)
        plsc.subcore_barrier()

        @pl.loop(0, nchunk)
        def _(j):
            pltpu.async_copy(hq_hbm.at[ridx_v.at[j]], rows_v, sem).wait()
            pltpu.sync_copy(rows_v, acc_sh.at[cidx_v.at[j]], add=True)

        plsc.subcore_barrier()
        pltpu.sync_copy(acc_sh.at[pl.ds(sid * rp, rp)], out_hbm.at[cid, sid])

    return edge_kernel


def _layer_block(h, g_row, gb, nd):
    p = jnp.sum(h * g_row, axis=1, keepdims=True) + gb
    return h * (jnp.tanh(p) * nd)


def _t1_body(deg_ref, x_ref, wt_ref, b_ref, g_ref, gb_ref,
             h_ref, hq_ref, nd_ref):
    deg = deg_ref[...]
    d = (deg[0] + deg[1])[:, 0:1]
    nd = lax.rsqrt(jnp.maximum(d, 1.0))
    h = jnp.dot(x_ref[...], wt_ref[...],
                preferred_element_type=jnp.float32) + b_ref[...]
    h_ref[...] = h
    hq_ref[...] = _layer_block(h, g_ref[...], gb_ref[0, 0], nd)
    nd_ref[...] = nd


def _t2_body(acc_ref, h1_ref, nd_ref, wt_ref, b_ref, g_ref, gb_ref,
             h2_ref, hq2_ref):
    acc = acc_ref[...]
    nd = nd_ref[...]
    o = nd * (acc[0] + acc[1]) + h1_ref[...]
    r = jnp.maximum(o, 0.0)
    h2 = jnp.dot(r, wt_ref[...], preferred_element_type=jnp.float32) + b_ref[...]
    h2_ref[...] = h2
    hq2_ref[...] = _layer_block(h2, g_ref[...], gb_ref[0, 0], nd)


def _t3_body(acc_ref, h2_ref, nd_ref, out_ref):
    acc = acc_ref[...]
    out_ref[...] = nd_ref[...] * (acc[0] + acc[1]) + h2_ref[...]


@functools.lru_cache(maxsize=None)
def _make_tc_kernels(n, d, blk):
    grid = (n // blk,)
    f32 = jnp.float32

    def bs(shape, imap):
        return pl.BlockSpec(shape, imap)

    row_map = lambda i: (i, 0)
    full_map = lambda i: (0, 0)

    t1 = pl.pallas_call(
        _t1_body,
        grid=grid,
        in_specs=[
            bs((2, blk, d), lambda i: (0, i, 0)),
            bs((blk, d), row_map),
            bs((d, d), full_map),
            bs((1, d), full_map),
            bs((1, d), full_map),
            bs((1, 1), full_map),
        ],
        out_specs=[
            bs((blk, d), row_map),
            bs((blk, d), row_map),
            bs((blk, 1), row_map),
        ],
        out_shape=[
            jax.ShapeDtypeStruct((n, d), f32),
            jax.ShapeDtypeStruct((n, d), f32),
            jax.ShapeDtypeStruct((n, 1), f32),
        ],
    )

    t2 = pl.pallas_call(
        _t2_body,
        grid=grid,
        in_specs=[
            bs((2, blk, d), lambda i: (0, i, 0)),
            bs((blk, d), row_map),
            bs((blk, 1), row_map),
            bs((d, d), full_map),
            bs((1, d), full_map),
            bs((1, d), full_map),
            bs((1, 1), full_map),
        ],
        out_specs=[
            bs((blk, d), row_map),
            bs((blk, d), row_map),
        ],
        out_shape=[
            jax.ShapeDtypeStruct((n, d), f32),
            jax.ShapeDtypeStruct((n, d), f32),
        ],
    )

    t3 = pl.pallas_call(
        _t3_body,
        grid=grid,
        in_specs=[
            bs((2, blk, d), lambda i: (0, i, 0)),
            bs((blk, d), row_map),
            bs((blk, 1), row_map),
        ],
        out_specs=bs((blk, d), row_map),
        out_shape=jax.ShapeDtypeStruct((n, d), f32),
    )

    return t1, t2, t3


def kernel(x, edge_index, W1, b1, G1, gb1, W2, b2, G2, gb2):
    n, d = x.shape
    e = edge_index.shape[1]
    per_w = -(-e // (NW * K)) * K
    nchunk = per_w // K
    e_pad = per_w * NW
    pad = e_pad - e

    ei = edge_index.astype(jnp.int32)
    row = ei[0]
    col = ei[1]
    row_deg = jnp.concatenate(
        [row, jnp.full((pad,), n, jnp.int32)]).reshape(NW, nchunk, K)
    row_feat = jnp.concatenate(
        [row, jnp.zeros((pad,), jnp.int32)]).reshape(NW, nchunk, K)
    col_pad = jnp.concatenate(
        [col, jnp.full((pad,), n, jnp.int32)]).reshape(NW, nchunk, K)

    ones_d = jnp.ones((K, d), jnp.float32)
    zeros_d = jnp.zeros((n // NS, d), jnp.float32)

    deg_k = _make_deg_kernel(n, d, nchunk)
    edge_k = _make_edge_kernel(n, d, nchunk)
    t1, t2, t3 = _make_tc_kernels(n, d, 400)

    deg2 = deg_k(row_deg, ones_d, zeros_d).reshape(NC, n, d)
    h1, hq1, nd = t1(deg2, x, W1.T, b1.reshape(1, d), G1,
                     gb1.reshape(1, 1))
    acc1 = edge_k(hq1, row_feat, col_pad, zeros_d).reshape(NC, n, d)
    h2, hq2 = t2(acc1, h1, nd, W2.T, b2.reshape(1, d), G2,
                 gb2.reshape(1, 1))
    acc2 = edge_k(hq2, row_feat, col_pad, zeros_d).reshape(NC, n, d)
    return t3(acc2, h2, nd)
